# Initial kernel scaffold; baseline (speedup 1.0000x reference)
import functools

import jax
import jax.numpy as jnp
from jax import lax
from jax.experimental import pallas as pl
from jax.experimental.pallas import tpu as pltpu
from jax.experimental.pallas import tpu_sc as plsc

_N = 100000
_E = 3200000
_NC = 2
_NS = 16
_NW = _NC * _NS
_EW = _E // _NW
_C = 10000
_BN = 10000


def _pre_body(x0, x1, x2, Wl, bl, Wn, We, bc, g0, g1, g2, s0, s1, s2):
    gs = (g0, g1, g2)
    ss = (s0, s1, s2)
    for t, xb in enumerate((x0, x1, x2)):
        h = jnp.dot(xb[...], Wl[t], preferred_element_type=jnp.float32) + bl[t]
        gs[t][...] = jnp.dot(h, We[t], preferred_element_type=jnp.float32)
        ss[t][...] = (
            jnp.dot(h, Wn[t], preferred_element_type=jnp.float32) + bc[t]
        )


def _post_body(s0, s1, s2, p0, p1, p2, y0, y1, y2):
    ys = (y0, y1, y2)
    for t, (sb, pb) in enumerate(((s0, p0), (s1, p1), (s2, p2))):
        agg = pb[0] + pb[1]
        ys[t][...] = jax.nn.sigmoid(jnp.maximum(sb[...] + agg, 0.0))


def _sc_edge_kernel(edges, g0, g1, g2, zeros, out0, out1, out2,
                    src_v, dst_v, msg_v, acc0, acc1, acc2, sem):
    cid = lax.axis_index("c")
    sid = lax.axis_index("s")

    @pl.when(sid == 0)
    def _init():
        pltpu.sync_copy(zeros, acc0)
        pltpu.sync_copy(zeros, acc1)
        pltpu.sync_copy(zeros, acc2)

    plsc.subcore_barrier()

    base_w = (cid * _NS + sid) * _EW
    for t, (g, acc) in enumerate(((g0, acc0), (g1, acc1), (g2, acc2))):
        def body(ci, _, t=t, g=g, acc=acc):
            base = base_w + ci * _C
            pltpu.sync_copy(edges.at[t, 0, pl.ds(base, _C)], src_v)
            pltpu.sync_copy(edges.at[t, 1, pl.ds(base, _C)], dst_v)
            pltpu.async_copy(g.at[src_v], msg_v, sem).wait()
            pltpu.sync_copy(msg_v, acc.at[dst_v], add=True)
            return 0
        lax.fori_loop(0, _EW // _C, body, 0)

    plsc.subcore_barrier()

    @pl.when(sid == 0)
    def _flush():
        pltpu.sync_copy(acc0, out0.at[cid])
        pltpu.sync_copy(acc1, out1.at[cid])
        pltpu.sync_copy(acc2, out2.at[cid])


def _make_sc_call(n):
    mesh = plsc.VectorSubcoreMesh(core_axis_name="c", subcore_axis_name="s")
    part = jax.ShapeDtypeStruct((_NC, n, 2), jnp.float32)
    return pl.kernel(
        _sc_edge_kernel,
        out_type=(part, part, part),
        mesh=mesh,
        scratch_types=[
            pltpu.VMEM((_C,), jnp.int32),
            pltpu.VMEM((_C,), jnp.int32),
            pltpu.VMEM((_C, 2), jnp.float32),
            pltpu.VMEM_SHARED((n, 2), jnp.float32),
            pltpu.VMEM_SHARED((n, 2), jnp.float32),
            pltpu.VMEM_SHARED((n, 2), jnp.float32),
            pltpu.SemaphoreType.DMA,
        ],
    )


def kernel(x0, x1, x2, edges, W_lin, b_lin, Wn, We, bc):
    n = x0.shape[0]
    grid = n // _BN

    full = lambda s: pl.BlockSpec(s, lambda i: (0,) * len(s))
    row = lambda c: pl.BlockSpec((_BN, c), lambda i: (i, 0))

    g0, g1, g2, s0, s1, s2 = pl.pallas_call(
        _pre_body,
        grid=(grid,),
        in_specs=[row(4), row(4), row(4),
                  full((3, 4, 4)), full((3, 4)),
                  full((3, 4, 2)), full((3, 4, 2)), full((3, 2))],
        out_specs=[row(2)] * 6,
        out_shape=[jax.ShapeDtypeStruct((n, 2), jnp.float32)] * 6,
    )(x0, x1, x2, W_lin, b_lin, Wn, We, bc)

    zeros = jnp.zeros((n, 2), jnp.float32)
    p0, p1, p2 = _make_sc_call(n)(edges, g0, g1, g2, zeros)

    prow = pl.BlockSpec((_NC, _BN, 2), lambda i: (0, i, 0))
    y0, y1, y2 = pl.pallas_call(
        _post_body,
        grid=(grid,),
        in_specs=[row(2)] * 3 + [prow] * 3,
        out_specs=[row(2)] * 3,
        out_shape=[jax.ShapeDtypeStruct((n, 2), jnp.float32)] * 3,
    )(s0, s1, s2, p0, p1, p2)

    return (y0, y1, y2)


# speedup vs baseline: 11.9963x; 11.9963x over previous
import functools

import jax
import jax.numpy as jnp
from jax import lax
from jax.experimental import pallas as pl
from jax.experimental.pallas import tpu as pltpu
from jax.experimental.pallas import tpu_sc as plsc

_N = 100000
_E = 3200000
_NC = 2
_NS = 16
_NW = _NC * _NS
_L = 128
_ROWS = _E // _L
_RW = _ROWS // _NW
_RREM = _ROWS - _RW * _NW
_RC = 71
_NCH = _RW // _RC


def _pre_body(x0, x1, x2, Wl, bl, Wn, We, bc, g0, g1, g2, s0, s1, s2):
    gs = (g0, g1, g2)
    ss = (s0, s1, s2)
    for t, xr in enumerate((x0, x1, x2)):
        x = xr[...]
        Wg = jnp.dot(Wl[t], We[t], preferred_element_type=jnp.float32)
        Ws = jnp.dot(Wl[t], Wn[t], preferred_element_type=jnp.float32)
        bg = jnp.dot(bl[t], We[t], preferred_element_type=jnp.float32)
        bs = (jnp.dot(bl[t], Wn[t],
                      preferred_element_type=jnp.float32) + bc[t])
        g = Wg[0][:, None] * x[0:1, :]
        s = Ws[0][:, None] * x[0:1, :]
        for k in range(1, 4):
            g = g + Wg[k][:, None] * x[k:k + 1, :]
            s = s + Ws[k][:, None] * x[k:k + 1, :]
        g = g + bg[:, None]
        gs[t][...] = jnp.concatenate(
            [g, jnp.zeros((6,) + g.shape[1:], jnp.float32)], axis=0)
        ss[t][...] = s + bs[:, None]


def _post_body(s0, s1, s2, p0, p1, p2, y0, y1, y2):
    ys = (y0, y1, y2)
    for t, (sb, pb) in enumerate(((s0, p0), (s1, p1), (s2, p2))):
        z = sb[...] + pb[0] + pb[1]
        ys[t][...] = jax.nn.sigmoid(jnp.maximum(z, 0.0))


def _sc_edge_kernel(edges, g0, g1, g2, zeros, out0, out1, out2,
                    src_row, dst_row, msg_v, acc, sem):
    cid = lax.axis_index("c")
    sid = lax.axis_index("s")
    wid = cid * _NS + sid

    @pl.when(sid == 0)
    def _init():
        pltpu.sync_copy(zeros, acc)

    plsc.subcore_barrier()

    def do_rows(t, g, row0, nrows_static):
        def row_body(j, _):
            pltpu.sync_copy(edges.at[2 * t * _ROWS + row0 + j], src_row)
            pltpu.sync_copy(edges.at[(2 * t + 1) * _ROWS + row0 + j],
                            dst_row)
            pltpu.async_copy(g.at[src_row], msg_v, sem).wait()
            pltpu.sync_copy(msg_v, acc.at[dst_row], add=True)
            return 0
        lax.fori_loop(0, nrows_static, row_body, 0)

    for t, (g, out) in enumerate(((g0, out0), (g1, out1), (g2, out2))):
        def chunk_body(ci, _, t=t, g=g):
            do_rows(t, g, wid * _RW + ci * _RC, _RC)
            return 0
        lax.fori_loop(0, _NCH, chunk_body, 0)

        @pl.when(wid < _RREM)
        def _tail(t=t, g=g):
            do_rows(t, g, _NW * _RW + wid, 1)

        plsc.subcore_barrier()

        @pl.when(sid == 0)
        def _flush(out=out):
            pltpu.sync_copy(acc, out.at[cid])
            if t < 2:
                pltpu.sync_copy(zeros, acc)

        plsc.subcore_barrier()


def _make_sc_call(n):
    mesh = plsc.VectorSubcoreMesh(core_axis_name="c", subcore_axis_name="s")
    part = jax.ShapeDtypeStruct((_NC, n, 8), jnp.float32)
    return pl.kernel(
        _sc_edge_kernel,
        out_type=(part, part, part),
        mesh=mesh,
        scratch_types=[
            pltpu.VMEM((_L,), jnp.int32),
            pltpu.VMEM((_L,), jnp.int32),
            pltpu.VMEM((_L, 8), jnp.float32),
            pltpu.VMEM_SHARED((n, 8), jnp.float32),
            pltpu.SemaphoreType.DMA,
        ],
        compiler_params=pltpu.CompilerParams(use_tc_tiling_on_sc=False),
    )


def kernel(x0, x1, x2, edges, W_lin, b_lin, Wn, We, bc):
    n = x0.shape[0]

    gT0, gT1, gT2, sT0, sT1, sT2 = pl.pallas_call(
        _pre_body,
        out_shape=[jax.ShapeDtypeStruct((8, n), jnp.float32)] * 3
        + [jax.ShapeDtypeStruct((2, n), jnp.float32)] * 3,
    )(x0.T, x1.T, x2.T, W_lin, b_lin, Wn, We, bc)

    zeros = jnp.zeros((n, 8), jnp.float32)
    p0, p1, p2 = _make_sc_call(n)(
        edges.reshape(-1, _L), gT0.T, gT1.T, gT2.T, zeros)

    yT0, yT1, yT2 = pl.pallas_call(
        _post_body,
        out_shape=[jax.ShapeDtypeStruct((2, n), jnp.float32)] * 3,
    )(sT0, sT1, sT2,
      p0[:, :, :2].transpose(0, 2, 1),
      p1[:, :, :2].transpose(0, 2, 1),
      p2[:, :, :2].transpose(0, 2, 1))

    return (yT0.T, yT1.T, yT2.T)


# speedup vs baseline: 21.3749x; 1.7818x over previous
import functools

import jax
import jax.numpy as jnp
from jax import lax
from jax.experimental import pallas as pl
from jax.experimental.pallas import tpu as pltpu
from jax.experimental.pallas import tpu_sc as plsc

_N = 100000
_E = 3200000
_NC = 2
_NS = 16
_NW = _NC * _NS
_L = 128
_ROWS = _E // _L
_RW = _ROWS // _NW
_RREM = _ROWS - _RW * _NW
_RC = 71
_NCH = _RW // _RC


def _pre_body(x0, x1, x2, Wl, bl, Wn, We, bc, g0, g1, g2, s0, s1, s2):
    gs = (g0, g1, g2)
    ss = (s0, s1, s2)
    for t, xr in enumerate((x0, x1, x2)):
        x = xr[...]
        Wg = jnp.dot(Wl[t], We[t], preferred_element_type=jnp.float32)
        Ws = jnp.dot(Wl[t], Wn[t], preferred_element_type=jnp.float32)
        bg = jnp.dot(bl[t], We[t], preferred_element_type=jnp.float32)
        bs = (jnp.dot(bl[t], Wn[t],
                      preferred_element_type=jnp.float32) + bc[t])
        g = Wg[0][:, None] * x[0:1, :]
        s = Ws[0][:, None] * x[0:1, :]
        for k in range(1, 4):
            g = g + Wg[k][:, None] * x[k:k + 1, :]
            s = s + Ws[k][:, None] * x[k:k + 1, :]
        g = g + bg[:, None]
        gs[t][...] = jnp.concatenate(
            [g, jnp.zeros((6,) + g.shape[1:], jnp.float32)], axis=0)
        ss[t][...] = s + bs[:, None]


def _post_body(s0, s1, s2, p0, p1, p2, y0, y1, y2):
    ys = (y0, y1, y2)
    for t, (sb, pb) in enumerate(((s0, p0), (s1, p1), (s2, p2))):
        z = sb[...] + pb[0] + pb[1]
        ys[t][...] = jax.nn.sigmoid(jnp.maximum(z, 0.0))


def _sc_edge_kernel(edges, g0, g1, g2, zeros, out0, out1, out2,
                    src_v, dst_v, msg_v, acc, sem):
    cid = lax.axis_index("c")
    sid = lax.axis_index("s")
    wid = cid * _NS + sid

    @pl.when(sid == 0)
    def _init():
        pltpu.sync_copy(zeros, acc)

    plsc.subcore_barrier()

    def do_rows(t, g, row0, nrows_static):
        pltpu.sync_copy(
            edges.at[pl.ds(2 * t * _ROWS + row0, nrows_static)],
            src_v.at[pl.ds(0, nrows_static)])
        pltpu.sync_copy(
            edges.at[pl.ds((2 * t + 1) * _ROWS + row0, nrows_static)],
            dst_v.at[pl.ds(0, nrows_static)])

        def row_body(j, _):
            pltpu.async_copy(g.at[src_v.at[j]], msg_v, sem).wait()
            pltpu.sync_copy(msg_v, acc.at[dst_v.at[j]], add=True)
            return 0
        lax.fori_loop(0, nrows_static, row_body, 0)

    for t, (g, out) in enumerate(((g0, out0), (g1, out1), (g2, out2))):
        def chunk_body(ci, _, t=t, g=g):
            do_rows(t, g, wid * _RW + ci * _RC, _RC)
            return 0
        lax.fori_loop(0, _NCH, chunk_body, 0)

        @pl.when(wid < _RREM)
        def _tail(t=t, g=g):
            do_rows(t, g, _NW * _RW + wid, 1)

        plsc.subcore_barrier()

        @pl.when(sid == 0)
        def _flush(out=out):
            pltpu.sync_copy(acc, out.at[cid])
            if t < 2:
                pltpu.sync_copy(zeros, acc)

        plsc.subcore_barrier()


def _make_sc_call(n):
    mesh = plsc.VectorSubcoreMesh(core_axis_name="c", subcore_axis_name="s")
    part = jax.ShapeDtypeStruct((_NC, n, 8), jnp.float32)
    return pl.kernel(
        _sc_edge_kernel,
        out_type=(part, part, part),
        mesh=mesh,
        scratch_types=[
            pltpu.VMEM((_RC, _L), jnp.int32),
            pltpu.VMEM((_RC, _L), jnp.int32),
            pltpu.VMEM((_L, 8), jnp.float32),
            pltpu.VMEM_SHARED((n, 8), jnp.float32),
            pltpu.SemaphoreType.DMA,
        ],
        compiler_params=pltpu.CompilerParams(use_tc_tiling_on_sc=False),
    )


def kernel(x0, x1, x2, edges, W_lin, b_lin, Wn, We, bc):
    n = x0.shape[0]

    gT0, gT1, gT2, sT0, sT1, sT2 = pl.pallas_call(
        _pre_body,
        out_shape=[jax.ShapeDtypeStruct((8, n), jnp.float32)] * 3
        + [jax.ShapeDtypeStruct((2, n), jnp.float32)] * 3,
    )(x0.T, x1.T, x2.T, W_lin, b_lin, Wn, We, bc)

    zeros = jnp.zeros((n, 8), jnp.float32)
    p0, p1, p2 = _make_sc_call(n)(
        edges.reshape(-1, _L), gT0.T, gT1.T, gT2.T, zeros)

    yT0, yT1, yT2 = pl.pallas_call(
        _post_body,
        out_shape=[jax.ShapeDtypeStruct((2, n), jnp.float32)] * 3,
    )(sT0, sT1, sT2,
      p0[:, :, :2].transpose(0, 2, 1),
      p1[:, :, :2].transpose(0, 2, 1),
      p2[:, :, :2].transpose(0, 2, 1))

    return (yT0.T, yT1.T, yT2.T)


# speedup vs baseline: 38.7535x; 1.8130x over previous
import functools

import jax
import jax.numpy as jnp
from jax import lax
from jax.experimental import pallas as pl
from jax.experimental.pallas import tpu as pltpu
from jax.experimental.pallas import tpu_sc as plsc

_N = 100000
_E = 3200000
_NC = 2
_NS = 16
_NW = _NC * _NS
_L = 128
_ROWS = _E // _L
_RW = _ROWS // _NW
_RREM = _ROWS - _RW * _NW
_RC = 71
_NCH = _RW // _RC
_NB = 8


def _pre_body(x0, x1, x2, Wl, bl, Wn, We, bc, g0, g1, g2, s0, s1, s2):
    gs = (g0, g1, g2)
    ss = (s0, s1, s2)
    for t, xr in enumerate((x0, x1, x2)):
        x = xr[...]
        Wg = jnp.dot(Wl[t], We[t], preferred_element_type=jnp.float32)
        Ws = jnp.dot(Wl[t], Wn[t], preferred_element_type=jnp.float32)
        bg = jnp.dot(bl[t], We[t], preferred_element_type=jnp.float32)
        bs = (jnp.dot(bl[t], Wn[t],
                      preferred_element_type=jnp.float32) + bc[t])
        g = Wg[0][:, None] * x[0:1, :]
        s = Ws[0][:, None] * x[0:1, :]
        for k in range(1, 4):
            g = g + Wg[k][:, None] * x[k:k + 1, :]
            s = s + Ws[k][:, None] * x[k:k + 1, :]
        g = g + bg[:, None]
        gs[t][...] = jnp.concatenate(
            [g, jnp.zeros((6,) + g.shape[1:], jnp.float32)], axis=0)
        ss[t][...] = s + bs[:, None]


def _post_body(s0, s1, s2, p0, p1, p2, y0, y1, y2):
    ys = (y0, y1, y2)
    for t, (sb, pb) in enumerate(((s0, p0), (s1, p1), (s2, p2))):
        z = sb[...] + pb[0] + pb[1]
        ys[t][...] = jax.nn.sigmoid(jnp.maximum(z, 0.0))


def _sc_edge_kernel(edges, g0, g1, g2, zeros, out0, out1, out2,
                    src_v, dst_v, msg_v, acc, sem, sem2):
    cid = lax.axis_index("c")
    sid = lax.axis_index("s")
    wid = cid * _NS + sid

    @pl.when(sid == 0)
    def _init():
        pltpu.sync_copy(zeros, acc)

    plsc.subcore_barrier()

    def do_rows(t, g, row0, nrows_static):
        pltpu.sync_copy(
            edges.at[pl.ds(2 * t * _ROWS + row0, nrows_static)],
            src_v.at[pl.ds(0, nrows_static)])
        pltpu.sync_copy(
            edges.at[pl.ds((2 * t + 1) * _ROWS + row0, nrows_static)],
            dst_v.at[pl.ds(0, nrows_static)])

        nfull = nrows_static - nrows_static % _NB

        def ring_body(i, _):
            base = i * _NB
            cps = [pltpu.async_copy(g.at[src_v.at[base + b]],
                                    msg_v.at[b], sem)
                   for b in range(_NB)]
            for cp in cps:
                cp.wait()
            scps = [pltpu.async_copy(msg_v.at[b],
                                     acc.at[dst_v.at[base + b]],
                                     sem2, add=True)
                    for b in range(_NB)]
            for cp in scps:
                cp.wait()
            return 0
        lax.fori_loop(0, nfull // _NB, ring_body, 0)

        def row_body(j, _):
            pltpu.async_copy(g.at[src_v.at[j]], msg_v.at[0], sem).wait()
            pltpu.sync_copy(msg_v.at[0], acc.at[dst_v.at[j]], add=True)
            return 0
        lax.fori_loop(nfull, nrows_static, row_body, 0)

    for t, (g, out) in enumerate(((g0, out0), (g1, out1), (g2, out2))):
        def chunk_body(ci, _, t=t, g=g):
            do_rows(t, g, wid * _RW + ci * _RC, _RC)
            return 0
        lax.fori_loop(0, _NCH, chunk_body, 0)

        @pl.when(wid < _RREM)
        def _tail(t=t, g=g):
            do_rows(t, g, _NW * _RW + wid, 1)

        plsc.subcore_barrier()

        @pl.when(sid == 0)
        def _flush(out=out):
            pltpu.sync_copy(acc, out.at[cid])
            if t < 2:
                pltpu.sync_copy(zeros, acc)

        plsc.subcore_barrier()


def _make_sc_call(n):
    mesh = plsc.VectorSubcoreMesh(core_axis_name="c", subcore_axis_name="s")
    part = jax.ShapeDtypeStruct((_NC, n, 8), jnp.float32)
    return pl.kernel(
        _sc_edge_kernel,
        out_type=(part, part, part),
        mesh=mesh,
        scratch_types=[
            pltpu.VMEM((_RC, _L), jnp.int32),
            pltpu.VMEM((_RC, _L), jnp.int32),
            pltpu.VMEM((_NB, _L, 8), jnp.float32),
            pltpu.VMEM_SHARED((n, 8), jnp.float32),
            pltpu.SemaphoreType.DMA,
            pltpu.SemaphoreType.DMA,
        ],
        compiler_params=pltpu.CompilerParams(use_tc_tiling_on_sc=False),
    )


def kernel(x0, x1, x2, edges, W_lin, b_lin, Wn, We, bc):
    n = x0.shape[0]

    gT0, gT1, gT2, sT0, sT1, sT2 = pl.pallas_call(
        _pre_body,
        out_shape=[jax.ShapeDtypeStruct((8, n), jnp.float32)] * 3
        + [jax.ShapeDtypeStruct((2, n), jnp.float32)] * 3,
    )(x0.T, x1.T, x2.T, W_lin, b_lin, Wn, We, bc)

    zeros = jnp.zeros((n, 8), jnp.float32)
    p0, p1, p2 = _make_sc_call(n)(
        edges.reshape(-1, _L), gT0.T, gT1.T, gT2.T, zeros)

    yT0, yT1, yT2 = pl.pallas_call(
        _post_body,
        out_shape=[jax.ShapeDtypeStruct((2, n), jnp.float32)] * 3,
    )(sT0, sT1, sT2,
      p0[:, :, :2].transpose(0, 2, 1),
      p1[:, :, :2].transpose(0, 2, 1),
      p2[:, :, :2].transpose(0, 2, 1))

    return (yT0.T, yT1.T, yT2.T)
